# baseline (speedup 1.0000x reference)
import functools

import jax
import jax.numpy as jnp
from jax import lax
from jax.experimental import pallas as pl
from jax.experimental.pallas import tpu as pltpu
from jax.experimental.pallas import tpu_sc as plsc

_CHUNK = 128


def kernel(head_index, rel_type, tail_index, node_emb, rel_emb):
    B = head_index.shape[0]
    D = node_emb.shape[1]

    info = plsc.get_sparse_core_info()
    nw = info.num_cores * info.num_subcores
    b_per_w = B // nw
    assert B % nw == 0 and b_per_w % _CHUNK == 0
    n_chunks = b_per_w // _CHUNK

    mesh = plsc.VectorSubcoreMesh(core_axis_name="c", subcore_axis_name="s")

    @functools.partial(
        pl.kernel,
        mesh=mesh,
        compiler_params=pltpu.CompilerParams(use_tc_tiling_on_sc=False),
        out_type=(
            jax.ShapeDtypeStruct((B, D), jnp.float32),
            jax.ShapeDtypeStruct((B, D), jnp.float32),
            jax.ShapeDtypeStruct((B, D), jnp.float32),
        ),
        scratch_types=[
            pltpu.VMEM((n_chunks, _CHUNK), jnp.int32),
            pltpu.VMEM((n_chunks, _CHUNK), jnp.int32),
            pltpu.VMEM((n_chunks, _CHUNK), jnp.int32),
            pltpu.VMEM((b_per_w, D), jnp.float32),
            pltpu.VMEM((b_per_w, D), jnp.float32),
            pltpu.VMEM((b_per_w, D), jnp.float32),
            pltpu.SemaphoreType.DMA,
            pltpu.SemaphoreType.DMA,
            pltpu.SemaphoreType.DMA,
        ],
    )
    def sc_gather(head_hbm, rel_hbm, tail_hbm, node_hbm, relemb_hbm,
                  head_out, rel_out, tail_out,
                  hidx, ridx, tidx, hrows, rrows, trows,
                  sem_h, sem_r, sem_t):
        wid = lax.axis_index("s") * info.num_cores + lax.axis_index("c")
        base = wid * b_per_w

        for j in range(n_chunks):
            off = pl.ds(base + j * _CHUNK, _CHUNK)
            pltpu.sync_copy(head_hbm.at[off], hidx.at[j])
            pltpu.sync_copy(rel_hbm.at[off], ridx.at[j])
            pltpu.sync_copy(tail_hbm.at[off], tidx.at[j])

        copies = []
        for j in range(n_chunks):
            dst = pl.ds(j * _CHUNK, _CHUNK)
            copies.append(pltpu.async_copy(
                node_hbm.at[hidx.at[j]], hrows.at[dst], sem_h))
            copies.append(pltpu.async_copy(
                node_hbm.at[tidx.at[j]], trows.at[dst], sem_t))
            copies.append(pltpu.async_copy(
                relemb_hbm.at[ridx.at[j]], rrows.at[dst], sem_r))
        for c in copies:
            c.wait()

        out_sl = pl.ds(base, b_per_w)
        pltpu.sync_copy(hrows, head_out.at[out_sl])
        pltpu.sync_copy(rrows, rel_out.at[out_sl])
        pltpu.sync_copy(trows, tail_out.at[out_sl])

    return sc_gather(head_index, rel_type, tail_index, node_emb, rel_emb)


# speedup vs baseline: 1.6810x; 1.6810x over previous
import functools

import jax
import jax.numpy as jnp
from jax import lax
from jax.experimental import pallas as pl
from jax.experimental.pallas import tpu as pltpu
from jax.experimental.pallas import tpu_sc as plsc

_CHUNK = 128
_UNROLL = 16


def kernel(head_index, rel_type, tail_index, node_emb, rel_emb):
    B = head_index.shape[0]
    D = node_emb.shape[1]
    R = rel_emb.shape[0]

    info = plsc.get_sparse_core_info()
    nw = info.num_cores * info.num_subcores
    b_per_w = B // nw
    half = b_per_w // 2
    assert B % nw == 0 and half % _UNROLL == 0

    mesh = plsc.VectorSubcoreMesh(core_axis_name="c", subcore_axis_name="s")

    @functools.partial(
        pl.kernel,
        mesh=mesh,
        compiler_params=pltpu.CompilerParams(use_tc_tiling_on_sc=True),
        out_type=(
            jax.ShapeDtypeStruct((B, D), jnp.float32),
            jax.ShapeDtypeStruct((B, D), jnp.float32),
            jax.ShapeDtypeStruct((B, D), jnp.float32),
        ),
        scratch_types=[
            pltpu.VMEM((b_per_w,), jnp.int32),
            pltpu.VMEM((b_per_w,), jnp.int32),
            pltpu.VMEM((b_per_w,), jnp.int32),
            pltpu.VMEM((half, D), jnp.float32),
            pltpu.VMEM((half, D), jnp.float32),
            pltpu.VMEM((half, D), jnp.float32),
            pltpu.SemaphoreType.DMA,
            pltpu.SemaphoreType.DMA,
            pltpu.SemaphoreType.DMA,
        ],
    )
    def sc_gather(head_hbm, rel_hbm, tail_hbm, node_hbm, relemb_hbm,
                  head_out, rel_out, tail_out,
                  hidx, tidx, ridx,
                  hrows, rrows, trows, sem_h, sem_t, sem_r):
        cid = lax.axis_index("c")
        sid = lax.axis_index("s")
        wid = sid * info.num_cores + cid
        base = wid * b_per_w
        sl = pl.ds(base, b_per_w)

        pltpu.sync_copy(head_hbm.at[sl], hidx)
        pltpu.sync_copy(tail_hbm.at[sl], tidx)
        pltpu.sync_copy(rel_hbm.at[sl], ridx)

        for h in range(2):
            hbase = h * half

            def issue(step, _, hbase=hbase):
                pos = step * _UNROLL
                hv = hidx[pl.ds(hbase + pos, _UNROLL)]
                tv = tidx[pl.ds(hbase + pos, _UNROLL)]
                rv = ridx[pl.ds(hbase + pos, _UNROLL)]
                for j in range(_UNROLL):
                    pltpu.async_copy(node_hbm.at[pl.ds(hv[j], 1)],
                                     hrows.at[pl.ds(pos + j, 1)], sem_h)
                    pltpu.async_copy(node_hbm.at[pl.ds(tv[j], 1)],
                                     trows.at[pl.ds(pos + j, 1)], sem_t)
                    pltpu.async_copy(relemb_hbm.at[pl.ds(rv[j], 1)],
                                     rrows.at[pl.ds(pos + j, 1)], sem_r)
                return 0

            lax.fori_loop(0, half // _UNROLL, issue, 0)

            out_sl = pl.ds(base + hbase, half)
            pltpu.make_async_copy(node_hbm.at[pl.ds(0, half)], hrows,
                                  sem_h).wait()
            pltpu.sync_copy(hrows, head_out.at[out_sl])
            pltpu.make_async_copy(node_hbm.at[pl.ds(0, half)], trows,
                                  sem_t).wait()
            pltpu.sync_copy(trows, tail_out.at[out_sl])
            pltpu.make_async_copy(node_hbm.at[pl.ds(0, half)], rrows,
                                  sem_r).wait()
            pltpu.sync_copy(rrows, rel_out.at[out_sl])

    return sc_gather(head_index, rel_type, tail_index, node_emb, rel_emb)
